# baseline (speedup 1.0000x reference)
import functools

import jax
import jax.numpy as jnp
from jax.experimental import pallas as pl
from jax.experimental.pallas import tpu as pltpu

NCLS = 32
EMBD = 64
HD = 4
DHD = 16
BLK = 10000
BLK2 = 10000


def _dg(a, b, ca, cb):
    return jax.lax.dot_general(
        a, b, (((ca,), (cb,)), ((), ())), preferred_element_type=jnp.float32)


def _dg16(a, b, ca, cb):
    return jax.lax.dot_general(
        a.astype(jnp.bfloat16), b.astype(jnp.bfloat16),
        (((ca,), (cb,)), ((), ())), preferred_element_type=jnp.float32)


def _pass1_body(x_ref, cls_ref, clsc_ref, sem_ref, Ws_ref, bs_ref, Wq_ref,
                bq_ref, Wk_ref, bk_ref, Wv_ref, bv_ref, Wo_ref, bo_ref,
                WrecA_ref, WrecB_ref, brec_ref, WgA_ref, WgB_ref, bg_ref,
                gamma_ref, beta_ref, fused_ref, q_s, wkvT_s, bkvf_s,
                m_s, d_s, o_s, sx_s, cnt_s, *, nblk, blk):
    i = pl.program_id(0)

    @pl.when(i == 0)
    def _init():
        q_s[...] = jnp.concatenate(
            [_dg(sem_ref[...], Wq_ref[...], 1, 1) + bq_ref[...],
             jnp.zeros((NCLS, EMBD), jnp.float32)], axis=1
        ).astype(jnp.bfloat16)
        wkvT_s[...] = jnp.concatenate(
            [_dg(Ws_ref[...], Wk_ref[...], 0, 1),
             _dg(Ws_ref[...], Wv_ref[...], 0, 1)], axis=1).astype(jnp.bfloat16)
        bkvf_s[...] = jnp.concatenate(
            [_dg(bs_ref[...], Wk_ref[...], 1, 1) + bk_ref[...],
             _dg(bs_ref[...], Wv_ref[...], 1, 1) + bv_ref[...]], axis=1)
        m_s[...] = jnp.full((NCLS, HD), -3e38, jnp.float32)
        d_s[...] = jnp.zeros((NCLS, EMBD), jnp.float32)
        o_s[...] = jnp.zeros((NCLS, EMBD), jnp.float32)
        sx_s[...] = jnp.zeros((NCLS, EMBD), jnp.float32)
        cnt_s[...] = jnp.zeros((NCLS, 1), jnp.float32)

    xb = x_ref[...]
    cls2 = cls_ref[0]
    clsc = clsc_ref[...]
    msk = (jax.lax.broadcasted_iota(jnp.int32, (NCLS, blk), 0)
           == cls2)
    onehot = msk.astype(jnp.bfloat16)
    onehotT = (jax.lax.broadcasted_iota(jnp.int32, (blk, NCLS), 1)
               == clsc).astype(jnp.bfloat16)

    xb16 = xb.astype(jnp.bfloat16)
    kv = _dg16(xb16, wkvT_s[...], 1, 0) + bkvf_s[...]

    qg = _dg16(onehotT, q_s[...], 1, 0)
    prod16 = (kv * qg).astype(jnp.bfloat16)
    lane128 = jax.lax.broadcasted_iota(jnp.int32, (2 * EMBD, HD), 0)
    head128 = jax.lax.broadcasted_iota(jnp.int32, (2 * EMBD, HD), 1)
    eselk = ((lane128 // DHD == head128) & (lane128 < EMBD)
             ).astype(jnp.float32)
    sT = _dg16(eselk, prod16, 0, 1) * 0.25

    masked = jnp.where(msk[None, :, :], sT[:, None, :], -3e38)
    mblk = jnp.transpose(jnp.max(masked, axis=2))
    m_old = m_s[...]
    m_new = jnp.maximum(m_old, mblk)
    scale = jnp.exp(m_old - m_new)
    m_s[...] = m_new
    zmat = jnp.concatenate([prod16, onehotT], axis=1)
    zsel = jnp.concatenate([eselk * 0.25, -m_new], axis=0).astype(jnp.bfloat16)
    w16 = jnp.exp(_dg16(zmat, zsel, 1, 0)).astype(jnp.bfloat16)
    wq = _dg16(w16, (lane128 // DHD % HD == head128), 1, 1)
    lmask = jax.lax.broadcasted_iota(jnp.int32, (1, 2 * EMBD), 1) < EMBD
    wrow = wq * jnp.where(lmask, 1.0, kv)
    rhs = jnp.concatenate([wrow.astype(jnp.bfloat16), xb16, onehotT], axis=1)
    seg = _dg16(onehot, rhs, 1, 0)
    scale_exp = _dg(scale, eselk[:EMBD, :], 1, 1)
    d_s[...] = d_s[...] * scale_exp + seg[:, :EMBD]
    o_s[...] = o_s[...] * scale_exp + seg[:, EMBD:2 * EMBD]
    sx_s[...] = sx_s[...] + seg[:, 2 * EMBD:3 * EMBD]
    eye32 = (jax.lax.broadcasted_iota(jnp.int32, (NCLS, NCLS), 0)
             == jax.lax.broadcasted_iota(jnp.int32, (NCLS, NCLS), 1))
    cnt_s[...] = cnt_s[...] + jnp.sum(
        jnp.where(eye32, seg[:, 3 * EMBD:], 0.0), axis=1, keepdims=True)

    @pl.when(i == nblk - 1)
    def _epilogue():
        att = o_s[...] / jnp.maximum(d_s[...], 1e-30)
        attout = _dg(att, Wo_ref[...], 1, 1) + bo_ref[...]
        new_fea = (_dg(sem_ref[...], WrecA_ref[...], 1, 1)
                   + _dg(attout, WrecB_ref[...], 1, 1) + brec_ref[...])
        cnt = cnt_s[...]
        old_fea = (_dg(sx_s[...], Ws_ref[...], 1, 1) + cnt * bs_ref[...]
                   ) / jnp.maximum(cnt, 1.0)
        glogit = (_dg(old_fea, WgA_ref[...], 1, 1)
                  + _dg(new_fea, WgB_ref[...], 1, 1) + bg_ref[...])
        gate = 1.0 / (1.0 + jnp.exp(-glogit))
        fused = gate * old_fea + (1.0 - gate) * new_fea
        mu = jnp.mean(fused, axis=-1, keepdims=True)
        var = jnp.mean((fused - mu) ** 2, axis=-1, keepdims=True)
        fused_ref[...] = ((fused - mu) * jax.lax.rsqrt(var + 1e-5)
                          * gamma_ref[...] + beta_ref[...])


def _pass2_body(x_ref, clsc_ref, fused_ref, out_ref, *, blk):
    onehotT = (jax.lax.broadcasted_iota(jnp.int32, (blk, NCLS), 1)
               == clsc_ref[...]).astype(jnp.float32)
    out_ref[...] = _dg(onehotT, fused_ref[...], 1, 0) * x_ref[...]


def _full(shape):
    return pl.BlockSpec(shape, lambda i: tuple(0 for _ in shape))


def _branch(x, sem, cls, Ws, bs, Wq, bq, Wk, bk, Wv, bv, Wo, bo,
            Wrec, brec, Wgate, bgate, gamma, beta):
    n = x.shape[0]
    blk = BLK
    nblk = n // blk
    assert nblk * blk == n
    cls = cls.astype(jnp.int32)
    cls3 = cls.reshape(nblk, 1, blk)
    clsc = cls.reshape(n, 1)
    r = lambda a: a.reshape(1, EMBD)
    WrecA, WrecB = Wrec[:, :EMBD], Wrec[:, EMBD:]
    WgA, WgB = Wgate[:, :EMBD], Wgate[:, EMBD:]

    w64 = _full((EMBD, EMBD))
    b1 = _full((1, EMBD))
    fused = pl.pallas_call(
        functools.partial(_pass1_body, nblk=nblk, blk=blk),
        grid=(nblk,),
        in_specs=[
            pl.BlockSpec((blk, EMBD), lambda i: (i, 0)),
            pl.BlockSpec((1, 1, blk), lambda i: (i, 0, 0)),
            pl.BlockSpec((blk, 1), lambda i: (i, 0)),
            _full((NCLS, EMBD)), w64, b1, w64, b1, w64, b1, w64, b1,
            w64, b1, w64, w64, b1, w64, w64, b1, b1, b1,
        ],
        out_specs=_full((NCLS, EMBD)),
        out_shape=jax.ShapeDtypeStruct((NCLS, EMBD), jnp.float32),
        scratch_shapes=[
            pltpu.VMEM((NCLS, 2 * EMBD), jnp.bfloat16),
            pltpu.VMEM((EMBD, 2 * EMBD), jnp.bfloat16),
            pltpu.VMEM((1, 2 * EMBD), jnp.float32),
            pltpu.VMEM((NCLS, HD), jnp.float32),
            pltpu.VMEM((NCLS, EMBD), jnp.float32),
            pltpu.VMEM((NCLS, EMBD), jnp.float32),
            pltpu.VMEM((NCLS, EMBD), jnp.float32),
            pltpu.VMEM((NCLS, 1), jnp.float32),
        ],
    )(x, cls3, clsc, sem, Ws, r(bs), Wq, r(bq), Wk, r(bk), Wv, r(bv), Wo,
      r(bo), WrecA, WrecB, r(brec), WgA, WgB, r(bgate), r(gamma), r(beta))

    blk2 = BLK2
    nblk2 = n // blk2
    upd = pl.pallas_call(
        functools.partial(_pass2_body, blk=blk2),
        grid=(nblk2,),
        in_specs=[
            pl.BlockSpec((blk2, EMBD), lambda i: (i, 0)),
            pl.BlockSpec((blk2, 1), lambda i: (i, 0)),
            _full((NCLS, EMBD)),
        ],
        out_specs=pl.BlockSpec((blk2, EMBD), lambda i: (i, 0)),
        out_shape=jax.ShapeDtypeStruct((n, EMBD), jnp.float32),
    )(x, clsc, fused)
    return upd


def kernel(v, c, v_sem, c_sem, v_class, c_class, Ws_v, bs_v, Ws_c, bs_c,
           Wq_v, bq_v, Wk_v, bk_v, Wv_v, bv_v, Wo_v, bo_v, Wq_c, bq_c,
           Wk_c, bk_c, Wv_c, bv_c, Wo_c, bo_c, Wrec_v, brec_v, Wrec_c,
           brec_c, Wgate_v, bgate_v, Wgate_c, bgate_c, gamma, beta):
    v_upd = _branch(v, v_sem, v_class, Ws_v, bs_v, Wq_v, bq_v, Wk_v, bk_v,
                    Wv_v, bv_v, Wo_v, bo_v, Wrec_v, brec_v, Wgate_v, bgate_v,
                    gamma, beta)
    c_upd = _branch(c, c_sem, c_class, Ws_c, bs_c, Wq_c, bq_c, Wk_c, bk_c,
                    Wv_c, bv_c, Wo_c, bo_c, Wrec_c, brec_c, Wgate_c, bgate_c,
                    gamma, beta)
    return (v_upd, c_upd)
